# baseline (speedup 1.0000x reference)
import functools

import jax
import jax.numpy as jnp
from jax.experimental import pallas as pl
from jax.experimental.pallas import tpu as pltpu

_NBUF = 3


def _gcn_ring_kernel(
    x_ref, w_ref, b_ref, adj_ref, out_ref, sup_ref, buf_ref, sem_ref, *, block_m
):
    i = pl.program_id(0)
    nblk = pl.num_programs(0)

    def _copy(blk, slot):
        return pltpu.make_async_copy(
            adj_ref.at[pl.ds(blk * block_m, block_m), :],
            buf_ref.at[slot],
            sem_ref.at[slot],
        )

    @pl.when(i == 0)
    def _():
        for k in range(_NBUF - 1):
            _copy(k, k).start()
        sup = jax.lax.dot_general(
            x_ref[...],
            w_ref[...],
            dimension_numbers=(((1,), (1,)), ((), ())),
            preferred_element_type=jnp.float32,
        )
        sup_ref[...] = (sup + b_ref[...]).astype(jnp.bfloat16)

    nxt = i + _NBUF - 1

    @pl.when(nxt < nblk)
    def _():
        _copy(nxt, nxt % _NBUF).start()

    slot = i % _NBUF
    _copy(i, slot).wait()
    out_ref[...] = jax.lax.dot_general(
        buf_ref[slot].astype(jnp.bfloat16),
        sup_ref[...],
        dimension_numbers=(((1,), (0,)), ((), ())),
        preferred_element_type=jnp.float32,
    )


@functools.partial(jax.jit, static_argnames=("block_m",))
def _gcn(x, adj, W, b, block_m):
    n, d_in = x.shape
    d_out = W.shape[0]
    b2 = b.reshape(1, d_out)
    grid = (adj.shape[0] // block_m,)
    return pl.pallas_call(
        functools.partial(_gcn_ring_kernel, block_m=block_m),
        grid=grid,
        in_specs=[
            pl.BlockSpec(
                (n, d_in), lambda i: (0, 0),
                pipeline_mode=pl.Buffered(buffer_count=1),
            ),
            pl.BlockSpec((d_out, d_in), lambda i: (0, 0)),
            pl.BlockSpec((1, d_out), lambda i: (0, 0)),
            pl.BlockSpec(memory_space=pl.ANY),
        ],
        out_specs=pl.BlockSpec((block_m, d_out), lambda i: (i, 0)),
        out_shape=jax.ShapeDtypeStruct((adj.shape[0], d_out), jnp.float32),
        scratch_shapes=[
            pltpu.VMEM((n, d_out), jnp.bfloat16),
            pltpu.VMEM((_NBUF, block_m, n), jnp.float32),
            pltpu.SemaphoreType.DMA((_NBUF,)),
        ],
    )(x, W, b2, adj)


def kernel(x, adj, W, b):
    m = adj.shape[0]
    for cand in (400, 200, 80, 40, 16, 8):
        if m % cand == 0:
            return _gcn(x, adj, W, b, cand)
    return _gcn(x, adj, W, b, m)


# speedup vs baseline: 1.0345x; 1.0345x over previous
import functools

import jax
import jax.numpy as jnp
from jax.experimental import pallas as pl
from jax.experimental.pallas import tpu as pltpu


def _gcn_block_kernel(x_ref, w_ref, b_ref, adj_ref, out_ref, sup_ref):
    @pl.when(pl.program_id(0) == 0)
    def _():
        sup = jax.lax.dot_general(
            x_ref[...],
            w_ref[...],
            dimension_numbers=(((1,), (1,)), ((), ())),
            preferred_element_type=jnp.float32,
        )
        sup_ref[...] = (sup + b_ref[...]).astype(jnp.bfloat16)

    adj_bf = adj_ref[...].astype(jnp.bfloat16)
    out_ref[...] = jax.lax.dot_general(
        adj_bf,
        sup_ref[...],
        dimension_numbers=(((1,), (0,)), ((), ())),
        preferred_element_type=jnp.float32,
    )


@functools.partial(jax.jit, static_argnames=("block_m",))
def _gcn(x, adj, W, b, block_m):
    n, d_in = x.shape
    d_out = W.shape[0]
    b2 = b.reshape(1, d_out)
    grid = (adj.shape[0] // block_m,)
    return pl.pallas_call(
        _gcn_block_kernel,
        grid=grid,
        in_specs=[
            pl.BlockSpec((n, d_in), lambda i: (0, 0)),
            pl.BlockSpec((d_out, d_in), lambda i: (0, 0)),
            pl.BlockSpec((1, d_out), lambda i: (0, 0)),
            pl.BlockSpec((block_m, n), lambda i: (i, 0)),
        ],
        out_specs=pl.BlockSpec((block_m, d_out), lambda i: (i, 0)),
        out_shape=jax.ShapeDtypeStruct((adj.shape[0], d_out), jnp.float32),
        scratch_shapes=[pltpu.VMEM((n, d_out), jnp.bfloat16)],
    )(x, W, b2, adj)


def kernel(x, adj, W, b):
    m = adj.shape[0]
    for cand in (400, 200, 80, 40, 16, 8):
        if m % cand == 0:
            return _gcn(x, adj, W, b, cand)
    return _gcn(x, adj, W, b, m)


# speedup vs baseline: 1.0346x; 1.0001x over previous
import functools

import jax
import jax.numpy as jnp
from jax.experimental import pallas as pl
from jax.experimental.pallas import tpu as pltpu


def _gcn_block_kernel(x_ref, w_ref, b_ref, adj_ref, out_ref, sup_ref):
    @pl.when(pl.program_id(0) == 0)
    def _():
        sup = jax.lax.dot_general(
            x_ref[...],
            w_ref[...],
            dimension_numbers=(((1,), (1,)), ((), ())),
            preferred_element_type=jnp.float32,
        )
        sup_ref[...] = (sup + b_ref[...]).astype(jnp.bfloat16)

    adj_bf = adj_ref[...]
    out_ref[...] = jax.lax.dot_general(
        adj_bf,
        sup_ref[...].astype(jnp.float32),
        dimension_numbers=(((1,), (0,)), ((), ())),
        preferred_element_type=jnp.float32,
    )


@functools.partial(jax.jit, static_argnames=("block_m",))
def _gcn(x, adj, W, b, block_m):
    n, d_in = x.shape
    d_out = W.shape[0]
    b2 = b.reshape(1, d_out)
    grid = (adj.shape[0] // block_m,)
    return pl.pallas_call(
        _gcn_block_kernel,
        grid=grid,
        in_specs=[
            pl.BlockSpec((n, d_in), lambda i: (0, 0)),
            pl.BlockSpec((d_out, d_in), lambda i: (0, 0)),
            pl.BlockSpec((1, d_out), lambda i: (0, 0)),
            pl.BlockSpec((block_m, n), lambda i: (i, 0)),
        ],
        out_specs=pl.BlockSpec((block_m, d_out), lambda i: (i, 0)),
        out_shape=jax.ShapeDtypeStruct((adj.shape[0], d_out), jnp.float32),
        scratch_shapes=[pltpu.VMEM((n, d_out), jnp.bfloat16)],
    )(x, W, b2, adj)


def kernel(x, adj, W, b):
    m = adj.shape[0]
    for cand in (400, 200, 80, 40, 16, 8):
        if m % cand == 0:
            return _gcn(x, adj, W, b, cand)
    return _gcn(x, adj, W, b, m)


# speedup vs baseline: 1.0347x; 1.0001x over previous
import functools

import jax
import jax.numpy as jnp
from jax.experimental import pallas as pl
from jax.experimental.pallas import tpu as pltpu


def _gcn_block_kernel(x_ref, w_ref, b_ref, adj_ref, out_ref, sup_ref):
    @pl.when(pl.program_id(0) == 0)
    def _():
        sup = jax.lax.dot_general(
            x_ref[...],
            w_ref[...],
            dimension_numbers=(((1,), (1,)), ((), ())),
            preferred_element_type=jnp.float32,
        )
        sup_ref[...] = sup + b_ref[...]

    adj_bf = adj_ref[...]
    out_ref[...] = jax.lax.dot_general(
        adj_bf,
        sup_ref[...],
        dimension_numbers=(((1,), (0,)), ((), ())),
        preferred_element_type=jnp.float32,
    )


@functools.partial(jax.jit, static_argnames=("block_m",))
def _gcn(x, adj, W, b, block_m):
    n, d_in = x.shape
    d_out = W.shape[0]
    b2 = b.reshape(1, d_out)
    grid = (adj.shape[0] // block_m,)
    return pl.pallas_call(
        _gcn_block_kernel,
        grid=grid,
        in_specs=[
            pl.BlockSpec((n, d_in), lambda i: (0, 0)),
            pl.BlockSpec((d_out, d_in), lambda i: (0, 0)),
            pl.BlockSpec((1, d_out), lambda i: (0, 0)),
            pl.BlockSpec((block_m, n), lambda i: (i, 0)),
        ],
        out_specs=pl.BlockSpec((block_m, d_out), lambda i: (i, 0)),
        out_shape=jax.ShapeDtypeStruct((adj.shape[0], d_out), jnp.float32),
        scratch_shapes=[pltpu.VMEM((n, d_out), jnp.float32)],
    )(x, W, b2, adj)


def kernel(x, adj, W, b):
    m = adj.shape[0]
    for cand in (400, 200, 80, 40, 16, 8):
        if m % cand == 0:
            return _gcn(x, adj, W, b, cand)
    return _gcn(x, adj, W, b, m)
